# baseline (speedup 1.0000x reference)
import functools

import jax
import jax.numpy as jnp
from jax import lax
from jax.experimental import pallas as pl
from jax.experimental.pallas import tpu as pltpu
from jax.experimental.pallas import tpu_sc as plsc

B = 4
CIN = 384
H = 200
W = 176
HW = H * W
HW16 = HW // 16
NUM_CLASS = 4
A = 6
N_POS = 512
N_NEG = 4096
M = N_POS + N_NEG

NC, NS = 2, 16
NW = NC * NS
SPW = M // NW
CH = 16
NCHUNK = SPW // CH
KV = CIN // 16
NDMA = CH * KV // 8


def _sc_gather(table, base_sp):
  mesh = plsc.VectorSubcoreMesh(
      core_axis_name="c", subcore_axis_name="s", num_cores=NC, num_subcores=NS)

  def body(table_h, base_h, out_h, g_v, idx_v, bsp_v, sem):
    wid = lax.axis_index("s") * NC + lax.axis_index("c")
    s0 = wid * SPW
    iota = lax.iota(jnp.int32, 16)

    def chunk(ci, carry):
      c0 = s0 + ci * CH
      pltpu.sync_copy(base_h.at[pl.ds(c0 * 16, CH * 16)], bsp_v)
      for sl in range(CH):
        bspl = bsp_v[pl.ds(sl * 16, 16)]
        for k in range(KV):
          idx_v[pl.ds(sl * CIN + k * 16, 16)] = bspl + (iota + k * 16) * HW
      cps = [
          pltpu.async_copy(
              table_h.at[idx_v.at[pl.ds(j * 128, 128)]],
              g_v.at[pl.ds(j * 128, 128)], sem)
          for j in range(NDMA)
      ]
      for cp in cps:
        cp.wait()
      pltpu.sync_copy(g_v, out_h.at[pl.ds(c0 * CIN, CH * CIN)])
      return carry

    lax.fori_loop(0, NCHUNK, chunk, 0)

  f = pl.kernel(
      body,
      out_type=jax.ShapeDtypeStruct((M * CIN,), jnp.float32),
      mesh=mesh,
      scratch_types=[
          pltpu.VMEM((CH * CIN,), jnp.float32),
          pltpu.VMEM((CH * CIN,), jnp.int32),
          pltpu.VMEM((CH * 16,), jnp.int32),
          pltpu.SemaphoreType.DMA,
      ],
  )
  return f(table, base_sp)


def _loss_body(x_ref, w_ref, b_ref, a_ref, lbl_ref, rl_ref, out_ref):
  x = x_ref[...]
  w = w_ref[...]
  logits = jnp.dot(x, w, preferred_element_type=jnp.float32,
                   precision=lax.Precision.HIGHEST) + b_ref[...]
  a_bc = a_ref[...]
  lane = lax.broadcasted_iota(jnp.int32, (M, 128), 1)
  cls = []
  for c in range(NUM_CLASS):
    sel = lane == (c * A + a_bc)
    cls.append(jnp.sum(jnp.where(sel, logits, 0.0), axis=1, keepdims=True))
  mx = jnp.maximum(jnp.maximum(cls[0], cls[1]), jnp.maximum(cls[2], cls[3]))
  se = (jnp.exp(cls[0] - mx) + jnp.exp(cls[1] - mx)
        + jnp.exp(cls[2] - mx) + jnp.exp(cls[3] - mx))
  lse = jnp.log(se) + mx
  lbl = lbl_ref[...]
  picked = sum(jnp.where(lbl == c, cls[c], 0.0) for c in range(NUM_CLASS))
  cls_loss = jnp.mean(lse - picked)

  logits_p = logits[N_NEG:, :]
  a_p = a_ref[...][N_NEG:, :]
  lane_p = lax.broadcasted_iota(jnp.int32, (N_POS, 128), 1)
  racc = jnp.zeros((), jnp.float32)
  for j in range(7):
    sel = lane_p == (NUM_CLASS * A + j * A + a_p)
    pj = jnp.sum(jnp.where(sel, logits_p, 0.0), axis=1, keepdims=True)
    d = pj - rl_ref[...][:, j:j + 1]
    ad = jnp.abs(d)
    racc = racc + jnp.sum(jnp.where(ad < 1.0, 0.5 * d * d, ad - 0.5))
  reg_loss = racc / (N_POS * 7)
  out_ref[...] = jnp.full((1, 1), cls_loss + 2.0 * reg_loss, jnp.float32)


def kernel(inputs, pos_batch_ids, pos_bbox_ids, neg_batch_ids, neg_bbox_ids,
           cls_labels, reg_labels, Wc, bc, Wr, br):
  all_b = jnp.concatenate([neg_batch_ids, pos_batch_ids]).astype(jnp.int32)
  all_t = jnp.concatenate([neg_bbox_ids, pos_bbox_ids]).astype(jnp.int32)
  a_sel = all_t // HW
  hw = all_t % HW
  base_elem = all_b * (CIN * HW) + hw
  base_sp = jnp.broadcast_to(base_elem[:, None], (M, 16)).reshape(M * 16)

  table = inputs.reshape(B * CIN * HW)
  x_flat = _sc_gather(table, base_sp)
  x_mat = x_flat.reshape(M, CIN)

  wp = jnp.concatenate(
      [Wc.T, Wr.T, jnp.zeros((CIN, 128 - (NUM_CLASS + 7) * A), jnp.float32)],
      axis=1)
  bp = jnp.concatenate(
      [bc, br, jnp.zeros((128 - (NUM_CLASS + 7) * A,), jnp.float32)]
  ).reshape(1, 128)
  rl_pad = jnp.concatenate(
      [reg_labels, jnp.zeros((N_POS, 1), jnp.float32)], axis=1)

  res = pl.pallas_call(
      _loss_body,
      out_shape=jax.ShapeDtypeStruct((1, 1), jnp.float32),
  )(x_mat, wp, bp, a_sel.reshape(M, 1), cls_labels.astype(jnp.int32).reshape(M, 1),
    rl_pad)
  return res[0, 0]
